# speedup vs baseline: 2.5276x; 2.5276x over previous
import functools

import jax
import jax.numpy as jnp
from jax import lax
from jax.experimental import pallas as pl
from jax.experimental.pallas import tpu as pltpu
from jax.experimental.pallas import tpu_sc as plsc

B, L, S, D = 1024, 200, 50, 128
NR = 6

NC, NS = 2, 16
NW = NC * NS
CH = 128

N_ITEM = B * L
N_USER_PAD = 53248
ITEM_PER_W = N_ITEM // NW
USER_PER_W = N_USER_PAD // NW


def _sc_gather_body(tab_i, tab_u, idx_i_hbm, idx_u_hbm, out_i, out_u,
                    idx_v, rows_v, sem):
    wid = lax.axis_index("s") * NC + lax.axis_index("c")

    def gather_loop(tab, idx_hbm, out, base, n_chunks):
        def body(j, carry):
            off = base + j * CH
            pltpu.sync_copy(idx_hbm.at[pl.ds(off, CH)], idx_v)
            pltpu.async_copy(tab.at[idx_v], rows_v, sem).wait()
            pltpu.sync_copy(rows_v, out.at[pl.ds(off, CH)])
            return carry
        lax.fori_loop(0, n_chunks, body, 0)

    gather_loop(tab_i, idx_i_hbm, out_i, wid * ITEM_PER_W, ITEM_PER_W // CH)
    gather_loop(tab_u, idx_u_hbm, out_u, wid * USER_PER_W, USER_PER_W // CH)


@jax.jit
def _sc_gather(embed_i_w, embed_u_w, idx_i, idx_u):
    mesh = plsc.VectorSubcoreMesh(core_axis_name="c", subcore_axis_name="s")
    return pl.kernel(
        _sc_gather_body,
        out_type=[
            jax.ShapeDtypeStruct((N_ITEM, D), jnp.float32),
            jax.ShapeDtypeStruct((N_USER_PAD, D), jnp.float32),
        ],
        mesh=mesh,
        scratch_types=[
            pltpu.VMEM((CH,), jnp.int32),
            pltpu.VMEM((CH, D), jnp.float32),
            pltpu.SemaphoreType.DMA,
        ],
    )(embed_i_w, embed_u_w, idx_i, idx_u)


BLK = 8
RL = BLK * L
RS = BLK * S


def _tc_body(qa_ref, ur_ref, un_ref, pi_ref,
             emb_r8_ref, gvw1a_ref, gvw1b_ref, gvb1_ref, gvw2_ref, gvb2_ref,
             aIw1a_ref, aIw1b_ref, aIb1_ref, aIw2_ref, aIb2_ref, aIw3_ref, aIb3_ref,
             aSw1a_ref, aSw1b_ref, aSb1_ref, aSw2_ref, aSb2_ref, aSw3_ref, aSb3_ref,
             mw1a_ref, mw1b_ref, mb1_ref, mw2_ref, mb2_ref,
             out_ref):
    f32 = jnp.float32
    cdim0 = (((0,), (0,)), ((), ()))

    qa = qa_ref[...]
    piB = pi_ref[...]

    onehot_r = (ur_ref[...] ==
                lax.broadcasted_iota(jnp.int32, (RL, 8), 1)).astype(f32)
    tr8 = jnp.dot(emb_r8_ref[...], gvw1b_ref[...],
                  preferred_element_type=f32)
    h = jnp.maximum(jnp.dot(qa, gvw1a_ref[...], preferred_element_type=f32)
                    + jnp.dot(onehot_r, tr8, preferred_element_type=f32)
                    + gvb1_ref[...], 0.0)
    xia = jnp.maximum(jnp.dot(h, gvw2_ref[...], preferred_element_type=f32)
                      + gvb2_ref[...], 0.0)

    def attention(feat, n_per_user, w1a_ref, w1b_ref, b1_ref, w2_ref, b2_ref,
                  w3_ref, b3_ref):
        n_rows = feat.shape[0]
        seg = (lax.broadcasted_iota(jnp.int32, (n_rows, BLK), 0) // n_per_user
               == lax.broadcasted_iota(jnp.int32, (n_rows, BLK), 1)).astype(f32)
        piW = jnp.dot(piB, w1b_ref[...], preferred_element_type=f32)
        a = jnp.maximum(jnp.dot(feat, w1a_ref[...], preferred_element_type=f32)
                        + jnp.dot(seg, piW, preferred_element_type=f32)
                        + b1_ref[...], 0.0)
        a = jnp.maximum(jnp.dot(a, w2_ref[...], preferred_element_type=f32)
                        + b2_ref[...], 0.0)
        logit = jnp.dot(a, w3_ref[...], preferred_element_type=f32) \
            + b3_ref[...]
        segmax = jnp.max(jnp.where(seg > 0.0, logit, -jnp.inf),
                         axis=0, keepdims=True)
        rowmax = jnp.sum(seg * segmax, axis=1, keepdims=True)
        e = jnp.exp(logit - rowmax)
        A = seg * e
        numer = lax.dot_general(A, feat, cdim0,
                                preferred_element_type=f32)
        ones = jnp.ones((n_rows, 1), f32)
        den = lax.dot_general(A, ones, cdim0,
                              preferred_element_type=f32)
        return numer / den

    hi_I = attention(xia, L, aIw1a_ref, aIw1b_ref, aIb1_ref, aIw2_ref,
                     aIb2_ref, aIw3_ref, aIb3_ref)
    hi_S = attention(un_ref[...], S, aSw1a_ref, aSw1b_ref, aSb1_ref, aSw2_ref,
                     aSb2_ref, aSw3_ref, aSb3_ref)

    h2 = jnp.maximum(jnp.dot(hi_I, mw1a_ref[...], preferred_element_type=f32)
                     + jnp.dot(hi_S, mw1b_ref[...], preferred_element_type=f32)
                     + mb1_ref[...], 0.0)
    out_ref[...] = jnp.maximum(
        jnp.dot(h2, mw2_ref[...], preferred_element_type=f32) + mb2_ref[...],
        0.0)


def _tc_compute(qa, ur2, un, pi, weights, interpret=False):
    n_blocks = B // BLK
    row_spec = pl.BlockSpec((RL, D), lambda b: (b, 0))
    ur_spec = pl.BlockSpec((RL, 1), lambda b: (b, 0))
    un_spec = pl.BlockSpec((RS, D), lambda b: (b, 0))
    pi_spec = pl.BlockSpec((BLK, D), lambda b: (b, 0))

    def w_spec(w):
        return pl.BlockSpec(w.shape, lambda b: tuple(0 for _ in w.shape))

    return pl.pallas_call(
        _tc_body,
        grid=(n_blocks,),
        in_specs=[row_spec, ur_spec, un_spec, pi_spec] +
                 [w_spec(w) for w in weights],
        out_specs=pl.BlockSpec((BLK, D), lambda b: (b, 0)),
        out_shape=jax.ShapeDtypeStruct((B, D), jnp.float32),
        compiler_params=pltpu.CompilerParams(
            dimension_semantics=("arbitrary",)),
        interpret=interpret,
    )(qa, ur2, un, pi, *weights)


def _prep_weights(embed_r_w, gv_w1, gv_b1, gv_w2, gv_b2,
                  attI_w1, attI_b1, attI_w2, attI_b2, attI_w3, attI_b3,
                  attS_w1, attS_b1, attS_w2, attS_b2, attS_w3, attS_b3,
                  mlp_w1, mlp_b1, mlp_w2, mlp_b2):
    emb_r8 = jnp.zeros((8, D), jnp.float32).at[:NR].set(embed_r_w)
    row = lambda v: v.reshape(1, -1)
    return [
        emb_r8, gv_w1[:D], gv_w1[D:], row(gv_b1), gv_w2, row(gv_b2),
        attI_w1[:D], attI_w1[D:], row(attI_b1), attI_w2, row(attI_b2),
        attI_w3, row(attI_b3),
        attS_w1[:D], attS_w1[D:], row(attS_b1), attS_w2, row(attS_b2),
        attS_w3, row(attS_b3),
        mlp_w1[:D], mlp_w1[D:], row(mlp_b1), mlp_w2, row(mlp_b2),
    ]


def kernel(nodes_u, history_u_lists_batch, social_adj_lists_batch,
           history_ur_lists_batch,
           embed_u_w, embed_i_w, embed_r_w,
           gv_w1, gv_b1, gv_w2, gv_b2,
           attI_w1, attI_b1, attI_w2, attI_b2, attI_w3, attI_b3,
           attS_w1, attS_b1, attS_w2, attS_b2, attS_w3, attS_b3,
           mlp_w1, mlp_b1, mlp_w2, mlp_b2):
    idx_i = history_u_lists_batch.reshape(N_ITEM)
    idx_u = jnp.concatenate([
        social_adj_lists_batch.reshape(B * S), nodes_u,
        jnp.zeros((N_USER_PAD - B * S - B,), jnp.int32)])
    qa, gathered_u = _sc_gather(embed_i_w, embed_u_w, idx_i, idx_u)
    un = gathered_u[:B * S]
    pi = gathered_u[B * S:B * S + B]

    ur2 = history_ur_lists_batch.reshape(N_ITEM, 1)
    weights = _prep_weights(
        embed_r_w, gv_w1, gv_b1, gv_w2, gv_b2,
        attI_w1, attI_b1, attI_w2, attI_b2, attI_w3, attI_b3,
        attS_w1, attS_b1, attS_w2, attS_b2, attS_w3, attS_b3,
        mlp_w1, mlp_b1, mlp_w2, mlp_b2)
    return _tc_compute(qa, ur2, un, pi, weights)


# speedup vs baseline: 2.7695x; 1.0957x over previous
import functools

import jax
import jax.numpy as jnp
from jax import lax
from jax.experimental import pallas as pl
from jax.experimental.pallas import tpu as pltpu
from jax.experimental.pallas import tpu_sc as plsc

B, L, S, D = 1024, 200, 50, 128
NR = 6

NC, NS = 2, 16
NW = NC * NS
CH = 128

N_ITEM = B * L
N_USER_PAD = 53248
ITEM_PER_W = N_ITEM // NW
USER_PER_W = N_USER_PAD // NW


def _sc_gather_body(tab_i, tab_u, idx_i_hbm, idx_u_hbm, out_i, out_u,
                    idx_v, rows_v, sem):
    wid = lax.axis_index("s") * NC + lax.axis_index("c")

    def gather_loop(tab, idx_hbm, out, base, n_chunks):
        def body(j, carry):
            off = base + j * CH
            pltpu.sync_copy(idx_hbm.at[pl.ds(off, CH)], idx_v)
            pltpu.async_copy(tab.at[idx_v], rows_v, sem).wait()
            pltpu.sync_copy(rows_v, out.at[pl.ds(off, CH)])
            return carry
        lax.fori_loop(0, n_chunks, body, 0)

    gather_loop(tab_i, idx_i_hbm, out_i, wid * ITEM_PER_W, ITEM_PER_W // CH)
    gather_loop(tab_u, idx_u_hbm, out_u, wid * USER_PER_W, USER_PER_W // CH)


@jax.jit
def _sc_gather(embed_i_w, embed_u_w, idx_i, idx_u):
    mesh = plsc.VectorSubcoreMesh(core_axis_name="c", subcore_axis_name="s")
    return pl.kernel(
        _sc_gather_body,
        out_type=[
            jax.ShapeDtypeStruct((N_ITEM, D), jnp.float32),
            jax.ShapeDtypeStruct((N_USER_PAD, D), jnp.float32),
        ],
        mesh=mesh,
        scratch_types=[
            pltpu.VMEM((CH,), jnp.int32),
            pltpu.VMEM((CH, D), jnp.float32),
            pltpu.SemaphoreType.DMA,
        ],
    )(embed_i_w, embed_u_w, idx_i, idx_u)


BLK = 8
RL = BLK * L
RS = BLK * S


def _tc_body(qa_ref, ur_ref, un_ref, pi_ref,
             emb_r8_ref, gvw1a_ref, gvw1b_ref, gvb1_ref, gvw2_ref, gvb2_ref,
             aIw1a_ref, aIw1b_ref, aIb1_ref, aIw2_ref, aIb2_ref, aIw3_ref, aIb3_ref,
             aSw1a_ref, aSw1b_ref, aSb1_ref, aSw2_ref, aSb2_ref, aSw3_ref, aSb3_ref,
             mw1a_ref, mw1b_ref, mb1_ref, mw2_ref, mb2_ref,
             out_ref):
    f32 = jnp.float32
    bf16 = jnp.bfloat16
    cdim0 = (((0,), (0,)), ((), ()))

    def mm(x, w):
        return jnp.dot(x.astype(bf16), w.astype(bf16),
                       preferred_element_type=f32)

    qa = qa_ref[...]
    piB = pi_ref[...]

    onehot_r = (ur_ref[...] ==
                lax.broadcasted_iota(jnp.int32, (RL, 8), 1)).astype(f32)
    tr8 = jnp.dot(emb_r8_ref[...], gvw1b_ref[...],
                  preferred_element_type=f32)
    h = jnp.maximum(mm(qa, gvw1a_ref[...])
                    + jnp.dot(onehot_r, tr8, preferred_element_type=f32)
                    + gvb1_ref[...], 0.0)
    xia = jnp.maximum(mm(h, gvw2_ref[...])
                      + gvb2_ref[...], 0.0)

    def attention(feat, n_per_user, w1a_ref, w1b_ref, b1_ref, w2_ref, b2_ref,
                  w3_ref, b3_ref):
        n_rows = feat.shape[0]
        seg = (lax.broadcasted_iota(jnp.int32, (n_rows, BLK), 0) // n_per_user
               == lax.broadcasted_iota(jnp.int32, (n_rows, BLK), 1)).astype(f32)
        piW = jnp.dot(piB, w1b_ref[...], preferred_element_type=f32)
        a = jnp.maximum(mm(feat, w1a_ref[...])
                        + jnp.dot(seg, piW, preferred_element_type=f32)
                        + b1_ref[...], 0.0)
        a = jnp.maximum(mm(a, w2_ref[...])
                        + b2_ref[...], 0.0)
        logit = jnp.dot(a, w3_ref[...], preferred_element_type=f32) \
            + b3_ref[...]
        segmax = jnp.max(jnp.where(seg > 0.0, logit, -jnp.inf),
                         axis=0, keepdims=True)
        rowmax = jnp.sum(seg * segmax, axis=1, keepdims=True)
        e = jnp.exp(logit - rowmax)
        A = seg * e
        numer = lax.dot_general(A, feat, cdim0,
                                preferred_element_type=f32)
        ones = jnp.ones((n_rows, 1), f32)
        den = lax.dot_general(A, ones, cdim0,
                              preferred_element_type=f32)
        return numer / den

    hi_I = attention(xia, L, aIw1a_ref, aIw1b_ref, aIb1_ref, aIw2_ref,
                     aIb2_ref, aIw3_ref, aIb3_ref)
    hi_S = attention(un_ref[...], S, aSw1a_ref, aSw1b_ref, aSb1_ref, aSw2_ref,
                     aSb2_ref, aSw3_ref, aSb3_ref)

    h2 = jnp.maximum(jnp.dot(hi_I, mw1a_ref[...], preferred_element_type=f32)
                     + jnp.dot(hi_S, mw1b_ref[...], preferred_element_type=f32)
                     + mb1_ref[...], 0.0)
    out_ref[...] = jnp.maximum(
        jnp.dot(h2, mw2_ref[...], preferred_element_type=f32) + mb2_ref[...],
        0.0)


def _tc_compute(qa, ur2, un, pi, weights, interpret=False):
    n_blocks = B // BLK
    row_spec = pl.BlockSpec((RL, D), lambda b: (b, 0))
    ur_spec = pl.BlockSpec((RL, 1), lambda b: (b, 0))
    un_spec = pl.BlockSpec((RS, D), lambda b: (b, 0))
    pi_spec = pl.BlockSpec((BLK, D), lambda b: (b, 0))

    def w_spec(w):
        return pl.BlockSpec(w.shape, lambda b: tuple(0 for _ in w.shape))

    return pl.pallas_call(
        _tc_body,
        grid=(n_blocks,),
        in_specs=[row_spec, ur_spec, un_spec, pi_spec] +
                 [w_spec(w) for w in weights],
        out_specs=pl.BlockSpec((BLK, D), lambda b: (b, 0)),
        out_shape=jax.ShapeDtypeStruct((B, D), jnp.float32),
        compiler_params=pltpu.CompilerParams(
            dimension_semantics=("arbitrary",)),
        interpret=interpret,
    )(qa, ur2, un, pi, *weights)


def _prep_weights(embed_r_w, gv_w1, gv_b1, gv_w2, gv_b2,
                  attI_w1, attI_b1, attI_w2, attI_b2, attI_w3, attI_b3,
                  attS_w1, attS_b1, attS_w2, attS_b2, attS_w3, attS_b3,
                  mlp_w1, mlp_b1, mlp_w2, mlp_b2):
    emb_r8 = jnp.zeros((8, D), jnp.float32).at[:NR].set(embed_r_w)
    row = lambda v: v.reshape(1, -1)
    return [
        emb_r8, gv_w1[:D], gv_w1[D:], row(gv_b1), gv_w2, row(gv_b2),
        attI_w1[:D], attI_w1[D:], row(attI_b1), attI_w2, row(attI_b2),
        attI_w3, row(attI_b3),
        attS_w1[:D], attS_w1[D:], row(attS_b1), attS_w2, row(attS_b2),
        attS_w3, row(attS_b3),
        mlp_w1[:D], mlp_w1[D:], row(mlp_b1), mlp_w2, row(mlp_b2),
    ]


def kernel(nodes_u, history_u_lists_batch, social_adj_lists_batch,
           history_ur_lists_batch,
           embed_u_w, embed_i_w, embed_r_w,
           gv_w1, gv_b1, gv_w2, gv_b2,
           attI_w1, attI_b1, attI_w2, attI_b2, attI_w3, attI_b3,
           attS_w1, attS_b1, attS_w2, attS_b2, attS_w3, attS_b3,
           mlp_w1, mlp_b1, mlp_w2, mlp_b2):
    idx_i = history_u_lists_batch.reshape(N_ITEM)
    idx_u = jnp.concatenate([
        social_adj_lists_batch.reshape(B * S), nodes_u,
        jnp.zeros((N_USER_PAD - B * S - B,), jnp.int32)])
    qa, gathered_u = _sc_gather(embed_i_w, embed_u_w, idx_i, idx_u)
    un = gathered_u[:B * S]
    pi = gathered_u[B * S:B * S + B]

    ur2 = history_ur_lists_batch.reshape(N_ITEM, 1)
    weights = _prep_weights(
        embed_r_w, gv_w1, gv_b1, gv_w2, gv_b2,
        attI_w1, attI_b1, attI_w2, attI_b2, attI_w3, attI_b3,
        attS_w1, attS_b1, attS_w2, attS_b2, attS_w3, attS_b3,
        mlp_w1, mlp_b1, mlp_w2, mlp_b2)
    return _tc_compute(qa, ur2, un, pi, weights)


# speedup vs baseline: 2.9949x; 1.0814x over previous
import functools

import jax
import jax.numpy as jnp
from jax import lax
from jax.experimental import pallas as pl
from jax.experimental.pallas import tpu as pltpu
from jax.experimental.pallas import tpu_sc as plsc

B, L, S, D = 1024, 200, 50, 128
NR = 6

NC, NS = 2, 16
NW = NC * NS
CH = 128

N_ITEM = B * L
N_USER_PAD = 53248
ITEM_PER_W = N_ITEM // NW
USER_PER_W = N_USER_PAD // NW


def _sc_gather_body(tab_i, tab_u, idx_i_hbm, idx_u_hbm, out_i, out_u,
                    idx_v, rows_v, sem):
    wid = lax.axis_index("s") * NC + lax.axis_index("c")

    def gather_loop(tab, idx_hbm, out, base, n_chunks):
        def body(j, carry):
            off = base + j * CH
            pltpu.sync_copy(idx_hbm.at[pl.ds(off, CH)], idx_v)
            pltpu.async_copy(tab.at[idx_v], rows_v, sem).wait()
            pltpu.sync_copy(rows_v, out.at[pl.ds(off, CH)])
            return carry
        lax.fori_loop(0, n_chunks, body, 0)

    gather_loop(tab_i, idx_i_hbm, out_i, wid * ITEM_PER_W, ITEM_PER_W // CH)
    gather_loop(tab_u, idx_u_hbm, out_u, wid * USER_PER_W, USER_PER_W // CH)


@jax.jit
def _sc_gather(embed_i_w, embed_u_w, idx_i, idx_u):
    mesh = plsc.VectorSubcoreMesh(core_axis_name="c", subcore_axis_name="s")
    return pl.kernel(
        _sc_gather_body,
        out_type=[
            jax.ShapeDtypeStruct((N_ITEM, D), jnp.float32),
            jax.ShapeDtypeStruct((N_USER_PAD, D), jnp.float32),
        ],
        mesh=mesh,
        scratch_types=[
            pltpu.VMEM((CH,), jnp.int32),
            pltpu.VMEM((CH, D), jnp.float32),
            pltpu.SemaphoreType.DMA,
        ],
    )(embed_i_w, embed_u_w, idx_i, idx_u)


BLK = 8
RL = BLK * L
RS = BLK * S


def _tc_body(qa_ref, ur_ref, un_ref, pi_ref,
             emb_r8_ref, gvw1a_ref, gvw1b_ref, gvb1_ref, gvw2_ref, gvb2_ref,
             aIw1a_ref, aIw1b_ref, aIb1_ref, aIw2_ref, aIb2_ref, aIw3_ref, aIb3_ref,
             aSw1a_ref, aSw1b_ref, aSb1_ref, aSw2_ref, aSb2_ref, aSw3_ref, aSb3_ref,
             mw1a_ref, mw1b_ref, mb1_ref, mw2_ref, mb2_ref,
             out_ref):
    f32 = jnp.float32
    bf16 = jnp.bfloat16

    def dt(lhs, rhs, lc, rc):
        return lax.dot_general(lhs.astype(bf16), rhs.astype(bf16),
                               (((lc,), (rc,)), ((), ())),
                               preferred_element_type=f32)

    qa = qa_ref[...]
    piB = pi_ref[...]
    urT = ur_ref[...].reshape(1, RL)

    onehotT = (urT == lax.broadcasted_iota(jnp.int32, (8, RL), 0)) \
        .astype(bf16)
    tr8 = jnp.dot(emb_r8_ref[...], gvw1b_ref[...],
                  preferred_element_type=f32)
    hT = jnp.maximum(dt(gvw1a_ref[...], qa, 0, 1)
                     + dt(tr8, onehotT, 0, 0)
                     + gvb1_ref[...], 0.0)
    xiaT = jnp.maximum(dt(gvw2_ref[...], hT, 0, 0)
                       + gvb2_ref[...], 0.0)

    def attention(featT_for_mlp, lc_feat, value_dot, segT,
                  w1a_ref, w1b_ref, b1_ref, w2_ref, b2_ref, w3_ref, b3_ref):
        piWT = dt(w1b_ref[...], piB, 0, 1)
        aT = jnp.maximum(dt(w1a_ref[...], featT_for_mlp, 0, lc_feat)
                         + dt(piWT, segT, 1, 0)
                         + b1_ref[...], 0.0)
        aT = jnp.maximum(dt(w2_ref[...], aT, 0, 0) + b2_ref[...], 0.0)
        logitT = dt(w3_ref[...], aT, 0, 0) + b3_ref[...]
        e = jnp.exp(logitT - jnp.max(logitT))
        AT = segT * e
        numer = value_dot(AT)
        den = lax.dot_general(AT, jnp.ones((AT.shape[1], 1), f32),
                              (((1,), (0,)), ((), ())),
                              preferred_element_type=f32)
        return numer / den

    segTI = segTI_const()
    segTS = segTS_const()
    hi_I = attention(xiaT, 0, lambda AT: dt(AT, xiaT, 1, 1), segTI,
                     aIw1a_ref, aIw1b_ref, aIb1_ref, aIw2_ref, aIb2_ref,
                     aIw3_ref, aIb3_ref)
    un = un_ref[...]
    hi_S = attention(un, 1, lambda AT: dt(AT, un, 1, 0), segTS,
                     aSw1a_ref, aSw1b_ref, aSb1_ref, aSw2_ref, aSb2_ref,
                     aSw3_ref, aSb3_ref)

    h2 = jnp.maximum(dt(hi_I, mw1a_ref[...], 1, 0)
                     + dt(hi_S, mw1b_ref[...], 1, 0)
                     + mb1_ref[...], 0.0)
    out_ref[...] = jnp.maximum(
        dt(h2, mw2_ref[...], 1, 0) + mb2_ref[...], 0.0)[:BLK]


def segTI_const():
    return (lax.broadcasted_iota(jnp.int32, (8, RL), 1) // L
            == lax.broadcasted_iota(jnp.int32, (8, RL), 0)).astype(jnp.float32)


def segTS_const():
    return (lax.broadcasted_iota(jnp.int32, (8, RS), 1) // S
            == lax.broadcasted_iota(jnp.int32, (8, RS), 0)).astype(jnp.float32)


def _tc_compute(qa, ur2, un, pi, weights, interpret=False):
    n_blocks = B // BLK
    row_spec = pl.BlockSpec((RL, D), lambda b: (b, 0))
    ur_spec = pl.BlockSpec((1, 1, RL), lambda b: (b, 0, 0))
    un_spec = pl.BlockSpec((RS, D), lambda b: (b, 0))
    pi_spec = pl.BlockSpec((BLK, D), lambda b: (b, 0))

    def w_spec(w):
        return pl.BlockSpec(w.shape, lambda b: tuple(0 for _ in w.shape))

    return pl.pallas_call(
        _tc_body,
        grid=(n_blocks,),
        in_specs=[row_spec, ur_spec, un_spec, pi_spec] +
                 [w_spec(w) for w in weights],
        out_specs=pl.BlockSpec((BLK, D), lambda b: (b, 0)),
        out_shape=jax.ShapeDtypeStruct((B, D), jnp.float32),
        compiler_params=pltpu.CompilerParams(
            dimension_semantics=("arbitrary",)),
        interpret=interpret,
    )(qa, ur2, un, pi, *weights)


def _prep_weights(embed_r_w, gv_w1, gv_b1, gv_w2, gv_b2,
                  attI_w1, attI_b1, attI_w2, attI_b2, attI_w3, attI_b3,
                  attS_w1, attS_b1, attS_w2, attS_b2, attS_w3, attS_b3,
                  mlp_w1, mlp_b1, mlp_w2, mlp_b2):
    emb_r8 = jnp.zeros((8, D), jnp.float32).at[:NR].set(embed_r_w)
    row = lambda v: v.reshape(1, -1)
    col = lambda v: v.reshape(-1, 1)
    return [
        emb_r8, gv_w1[:D], gv_w1[D:], col(gv_b1), gv_w2, col(gv_b2),
        attI_w1[:D], attI_w1[D:], col(attI_b1), attI_w2, col(attI_b2),
        attI_w3, row(attI_b3),
        attS_w1[:D], attS_w1[D:], col(attS_b1), attS_w2, col(attS_b2),
        attS_w3, row(attS_b3),
        mlp_w1[:D], mlp_w1[D:], row(mlp_b1), mlp_w2, row(mlp_b2),
    ]


def kernel(nodes_u, history_u_lists_batch, social_adj_lists_batch,
           history_ur_lists_batch,
           embed_u_w, embed_i_w, embed_r_w,
           gv_w1, gv_b1, gv_w2, gv_b2,
           attI_w1, attI_b1, attI_w2, attI_b2, attI_w3, attI_b3,
           attS_w1, attS_b1, attS_w2, attS_b2, attS_w3, attS_b3,
           mlp_w1, mlp_b1, mlp_w2, mlp_b2):
    idx_i = history_u_lists_batch.reshape(N_ITEM)
    idx_u = jnp.concatenate([
        social_adj_lists_batch.reshape(B * S), nodes_u,
        jnp.zeros((N_USER_PAD - B * S - B,), jnp.int32)])
    qa, gathered_u = _sc_gather(embed_i_w, embed_u_w, idx_i, idx_u)
    un = gathered_u[:B * S]
    pi = gathered_u[B * S:B * S + B]

    ur2 = history_ur_lists_batch.reshape(B // BLK, 1, RL)
    weights = _prep_weights(
        embed_r_w, gv_w1, gv_b1, gv_w2, gv_b2,
        attI_w1, attI_b1, attI_w2, attI_b2, attI_w3, attI_b3,
        attS_w1, attS_b1, attS_w2, attS_b2, attS_w3, attS_b3,
        mlp_w1, mlp_b1, mlp_w2, mlp_b2)
    return _tc_compute(qa, ur2, un, pi, weights)


# speedup vs baseline: 3.8225x; 1.2763x over previous
import functools

import jax
import jax.numpy as jnp
from jax import lax
from jax.experimental import pallas as pl
from jax.experimental.pallas import tpu as pltpu
from jax.experimental.pallas import tpu_sc as plsc

B, L, S, D = 1024, 200, 50, 128
NR = 6

NC, NS = 2, 16
NW = NC * NS
CH = 128

N_ITEM = B * L
N_USER_PAD = 53248
ITEM_PER_W = N_ITEM // NW
USER_PER_W = N_USER_PAD // NW


def _sc_gather_body(tab_i, tab_u, idx_i_hbm, idx_u_hbm, out_i, out_u,
                    idx_v, rows_v, sem):
    wid = lax.axis_index("s") * NC + lax.axis_index("c")

    def gather_loop(tab, idx_hbm, out, base, n_chunks):
        def body(j, carry):
            off = base + j * CH
            pltpu.sync_copy(idx_hbm.at[pl.ds(off, CH)], idx_v)
            pltpu.async_copy(tab.at[idx_v], rows_v, sem).wait()
            pltpu.sync_copy(rows_v, out.at[pl.ds(off, CH)])
            return carry
        lax.fori_loop(0, n_chunks, body, 0)

    gather_loop(tab_i, idx_i_hbm, out_i, wid * ITEM_PER_W, ITEM_PER_W // CH)
    gather_loop(tab_u, idx_u_hbm, out_u, wid * USER_PER_W, USER_PER_W // CH)


@jax.jit
def _sc_gather(embed_i_w, embed_u_w, idx_i, idx_u):
    mesh = plsc.VectorSubcoreMesh(core_axis_name="c", subcore_axis_name="s")
    return pl.kernel(
        _sc_gather_body,
        out_type=[
            jax.ShapeDtypeStruct((N_ITEM, D), jnp.float32),
            jax.ShapeDtypeStruct((N_USER_PAD, D), jnp.float32),
        ],
        mesh=mesh,
        scratch_types=[
            pltpu.VMEM((CH,), jnp.int32),
            pltpu.VMEM((CH, D), jnp.float32),
            pltpu.SemaphoreType.DMA,
        ],
    )(embed_i_w, embed_u_w, idx_i, idx_u)


BLK = 16
RL = BLK * L
RS = BLK * S


def _tc_body(qa_ref, ur_ref, un_ref, pi_ref,
             emb_r8_ref, gvw1a_ref, gvw1b_ref, gvb1_ref, gvw2_ref, gvb2_ref,
             aIw1a_ref, aIw1b_ref, aIb1_ref, aIw2_ref, aIb2_ref, aIw3_ref, aIb3_ref,
             aSw1a_ref, aSw1b_ref, aSb1_ref, aSw2_ref, aSb2_ref, aSw3_ref, aSb3_ref,
             mw1a_ref, mw1b_ref, mb1_ref, mw2_ref, mb2_ref,
             out_ref):
    f32 = jnp.float32
    bf16 = jnp.bfloat16

    def dt(lhs, rhs, lc, rc):
        return lax.dot_general(lhs.astype(bf16), rhs.astype(bf16),
                               (((lc,), (rc,)), ((), ())),
                               preferred_element_type=f32)

    qa = qa_ref[...]
    piB = pi_ref[...]
    urT = ur_ref[...].reshape(1, RL)

    onehotT = (urT == lax.broadcasted_iota(jnp.int32, (8, RL), 0)) \
        .astype(bf16)
    tr8 = jnp.dot(emb_r8_ref[...], gvw1b_ref[...],
                  preferred_element_type=f32)
    hT = jnp.maximum(dt(gvw1a_ref[...], qa, 0, 1)
                     + dt(tr8, onehotT, 0, 0)
                     + gvb1_ref[...], 0.0)
    xiaT = jnp.maximum(dt(gvw2_ref[...], hT, 0, 0)
                       + gvb2_ref[...], 0.0)

    def attention(featT_for_mlp, lc_feat, value_dot, segT,
                  w1a_ref, w1b_ref, b1_ref, w2_ref, b2_ref, w3_ref, b3_ref):
        piWT = dt(w1b_ref[...], piB, 0, 1)
        aT = jnp.maximum(dt(w1a_ref[...], featT_for_mlp, 0, lc_feat)
                         + dt(piWT, segT, 1, 0)
                         + b1_ref[...], 0.0)
        aT = jnp.maximum(dt(w2_ref[...], aT, 0, 0) + b2_ref[...], 0.0)
        logitT = dt(w3_ref[...], aT, 0, 0) + b3_ref[...]
        e = jnp.exp(logitT - jnp.max(logitT))
        AT = segT * e
        numer = value_dot(AT)
        den = lax.dot_general(AT, jnp.ones((AT.shape[1], 1), f32),
                              (((1,), (0,)), ((), ())),
                              preferred_element_type=f32)
        return numer / den

    segTI = segTI_const()
    segTS = segTS_const()
    hi_I = attention(xiaT, 0, lambda AT: dt(AT, xiaT, 1, 1), segTI,
                     aIw1a_ref, aIw1b_ref, aIb1_ref, aIw2_ref, aIb2_ref,
                     aIw3_ref, aIb3_ref)
    un = un_ref[...]
    hi_S = attention(un, 1, lambda AT: dt(AT, un, 1, 0), segTS,
                     aSw1a_ref, aSw1b_ref, aSb1_ref, aSw2_ref, aSb2_ref,
                     aSw3_ref, aSb3_ref)

    h2 = jnp.maximum(dt(hi_I, mw1a_ref[...], 1, 0)
                     + dt(hi_S, mw1b_ref[...], 1, 0)
                     + mb1_ref[...], 0.0)
    out_ref[...] = jnp.maximum(
        dt(h2, mw2_ref[...], 1, 0) + mb2_ref[...], 0.0)[:BLK]


def segTI_const():
    return (lax.broadcasted_iota(jnp.int32, (BLK, RL), 1) // L
            == lax.broadcasted_iota(jnp.int32, (BLK, RL), 0)).astype(jnp.float32)


def segTS_const():
    return (lax.broadcasted_iota(jnp.int32, (BLK, RS), 1) // S
            == lax.broadcasted_iota(jnp.int32, (BLK, RS), 0)).astype(jnp.float32)


def _tc_compute(qa, ur2, un, pi, weights, interpret=False):
    n_blocks = B // BLK
    row_spec = pl.BlockSpec((RL, D), lambda b: (b, 0))
    ur_spec = pl.BlockSpec((1, 1, RL), lambda b: (b, 0, 0))
    un_spec = pl.BlockSpec((RS, D), lambda b: (b, 0))
    pi_spec = pl.BlockSpec((BLK, D), lambda b: (b, 0))

    def w_spec(w):
        return pl.BlockSpec(w.shape, lambda b: tuple(0 for _ in w.shape))

    return pl.pallas_call(
        _tc_body,
        grid=(n_blocks,),
        in_specs=[row_spec, ur_spec, un_spec, pi_spec] +
                 [w_spec(w) for w in weights],
        out_specs=pl.BlockSpec((BLK, D), lambda b: (b, 0)),
        out_shape=jax.ShapeDtypeStruct((B, D), jnp.float32),
        compiler_params=pltpu.CompilerParams(
            dimension_semantics=("arbitrary",)),
        interpret=interpret,
    )(qa, ur2, un, pi, *weights)


def _prep_weights(embed_r_w, gv_w1, gv_b1, gv_w2, gv_b2,
                  attI_w1, attI_b1, attI_w2, attI_b2, attI_w3, attI_b3,
                  attS_w1, attS_b1, attS_w2, attS_b2, attS_w3, attS_b3,
                  mlp_w1, mlp_b1, mlp_w2, mlp_b2):
    emb_r8 = jnp.zeros((8, D), jnp.float32).at[:NR].set(embed_r_w)
    row = lambda v: v.reshape(1, -1)
    col = lambda v: v.reshape(-1, 1)
    return [
        emb_r8, gv_w1[:D], gv_w1[D:], col(gv_b1), gv_w2, col(gv_b2),
        attI_w1[:D], attI_w1[D:], col(attI_b1), attI_w2, col(attI_b2),
        attI_w3, row(attI_b3),
        attS_w1[:D], attS_w1[D:], col(attS_b1), attS_w2, col(attS_b2),
        attS_w3, row(attS_b3),
        mlp_w1[:D], mlp_w1[D:], row(mlp_b1), mlp_w2, row(mlp_b2),
    ]


def kernel(nodes_u, history_u_lists_batch, social_adj_lists_batch,
           history_ur_lists_batch,
           embed_u_w, embed_i_w, embed_r_w,
           gv_w1, gv_b1, gv_w2, gv_b2,
           attI_w1, attI_b1, attI_w2, attI_b2, attI_w3, attI_b3,
           attS_w1, attS_b1, attS_w2, attS_b2, attS_w3, attS_b3,
           mlp_w1, mlp_b1, mlp_w2, mlp_b2):
    idx_i = history_u_lists_batch.reshape(N_ITEM)
    idx_u = jnp.concatenate([
        social_adj_lists_batch.reshape(B * S), nodes_u,
        jnp.zeros((N_USER_PAD - B * S - B,), jnp.int32)])
    qa, gathered_u = _sc_gather(embed_i_w, embed_u_w, idx_i, idx_u)
    un = gathered_u[:B * S]
    pi = gathered_u[B * S:B * S + B]

    ur2 = history_ur_lists_batch.reshape(B // BLK, 1, RL)
    weights = _prep_weights(
        embed_r_w, gv_w1, gv_b1, gv_w2, gv_b2,
        attI_w1, attI_b1, attI_w2, attI_b2, attI_w3, attI_b3,
        attS_w1, attS_b1, attS_w2, attS_b2, attS_w3, attS_b3,
        mlp_w1, mlp_b1, mlp_w2, mlp_b2)
    return _tc_compute(qa, ur2, un, pi, weights)


# speedup vs baseline: 4.3676x; 1.1426x over previous
import functools

import jax
import jax.numpy as jnp
from jax import lax
from jax.experimental import pallas as pl
from jax.experimental.pallas import tpu as pltpu
from jax.experimental.pallas import tpu_sc as plsc

B, L, S, D = 1024, 200, 50, 128
NR = 6

NC, NS = 2, 16
NW = NC * NS
CH = 128

N_ITEM = B * L
N_USER_PAD = 53248
ITEM_PER_W = N_ITEM // NW
USER_PER_W = N_USER_PAD // NW


def _sc_gather_body(tab_i, tab_u, idx_i_hbm, idx_u_hbm, out_i, out_u,
                    idx_v, rows_v, sem):
    wid = lax.axis_index("s") * NC + lax.axis_index("c")

    def gather_loop(tab, idx_hbm, out, base, n_chunks):
        def body(j, carry):
            off = base + j * CH
            pltpu.sync_copy(idx_hbm.at[pl.ds(off, CH)], idx_v)
            pltpu.async_copy(tab.at[idx_v], rows_v, sem).wait()
            pltpu.sync_copy(rows_v, out.at[pl.ds(off, CH)])
            return carry
        lax.fori_loop(0, n_chunks, body, 0)

    gather_loop(tab_i, idx_i_hbm, out_i, wid * ITEM_PER_W, ITEM_PER_W // CH)
    gather_loop(tab_u, idx_u_hbm, out_u, wid * USER_PER_W, USER_PER_W // CH)


@jax.jit
def _sc_gather(embed_i_w, embed_u_w, idx_i, idx_u):
    mesh = plsc.VectorSubcoreMesh(core_axis_name="c", subcore_axis_name="s")
    return pl.kernel(
        _sc_gather_body,
        out_type=[
            jax.ShapeDtypeStruct((N_ITEM, D), jnp.float32),
            jax.ShapeDtypeStruct((N_USER_PAD, D), jnp.float32),
        ],
        mesh=mesh,
        scratch_types=[
            pltpu.VMEM((CH,), jnp.int32),
            pltpu.VMEM((CH, D), jnp.float32),
            pltpu.SemaphoreType.DMA,
        ],
    )(embed_i_w, embed_u_w, idx_i, idx_u)


BLK = 32
RL = BLK * L
RS = BLK * S


def _tc_body(qa_ref, ur_ref, un_ref, pi_ref,
             emb_r8_ref, gvw1a_ref, gvw1b_ref, gvb1_ref, gvw2_ref, gvb2_ref,
             aIw1a_ref, aIw1b_ref, aIb1_ref, aIw2_ref, aIb2_ref, aIw3_ref, aIb3_ref,
             aSw1a_ref, aSw1b_ref, aSb1_ref, aSw2_ref, aSb2_ref, aSw3_ref, aSb3_ref,
             mw1a_ref, mw1b_ref, mb1_ref, mw2_ref, mb2_ref,
             out_ref):
    f32 = jnp.float32
    bf16 = jnp.bfloat16

    def dt(lhs, rhs, lc, rc):
        return lax.dot_general(lhs.astype(bf16), rhs.astype(bf16),
                               (((lc,), (rc,)), ((), ())),
                               preferred_element_type=f32)

    qa = qa_ref[...]
    piB = pi_ref[...]
    urT = ur_ref[...].reshape(1, RL)

    onehotT = (urT == lax.broadcasted_iota(jnp.int32, (8, RL), 0)) \
        .astype(bf16)
    tr8 = jnp.dot(emb_r8_ref[...], gvw1b_ref[...],
                  preferred_element_type=f32)
    hT = jnp.maximum(dt(gvw1a_ref[...], qa, 0, 1)
                     + dt(tr8, onehotT, 0, 0)
                     + gvb1_ref[...], 0.0)
    xiaT = jnp.maximum(dt(gvw2_ref[...], hT, 0, 0)
                       + gvb2_ref[...], 0.0)

    def attention(featT_for_mlp, lc_feat, value_dot, segT,
                  w1a_ref, w1b_ref, b1_ref, w2_ref, b2_ref, w3_ref, b3_ref):
        piWT = dt(w1b_ref[...], piB, 0, 1)
        aT = jnp.maximum(dt(w1a_ref[...], featT_for_mlp, 0, lc_feat)
                         + dt(piWT, segT, 1, 0)
                         + b1_ref[...], 0.0)
        aT = jnp.maximum(dt(w2_ref[...], aT, 0, 0) + b2_ref[...], 0.0)
        logitT = dt(w3_ref[...], aT, 0, 0) + b3_ref[...]
        e = jnp.exp(logitT - jnp.max(logitT))
        AT = segT * e
        numer = value_dot(AT)
        den = lax.dot_general(AT, jnp.ones((AT.shape[1], 1), f32),
                              (((1,), (0,)), ((), ())),
                              preferred_element_type=f32)
        return numer / den

    segTI = segTI_const()
    segTS = segTS_const()
    hi_I = attention(xiaT, 0, lambda AT: dt(AT, xiaT, 1, 1), segTI,
                     aIw1a_ref, aIw1b_ref, aIb1_ref, aIw2_ref, aIb2_ref,
                     aIw3_ref, aIb3_ref)
    un = un_ref[...]
    hi_S = attention(un, 1, lambda AT: dt(AT, un, 1, 0), segTS,
                     aSw1a_ref, aSw1b_ref, aSb1_ref, aSw2_ref, aSb2_ref,
                     aSw3_ref, aSb3_ref)

    h2 = jnp.maximum(dt(hi_I, mw1a_ref[...], 1, 0)
                     + dt(hi_S, mw1b_ref[...], 1, 0)
                     + mb1_ref[...], 0.0)
    out_ref[...] = jnp.maximum(
        dt(h2, mw2_ref[...], 1, 0) + mb2_ref[...], 0.0)[:BLK]


def segTI_const():
    return (lax.broadcasted_iota(jnp.int32, (BLK, RL), 1) // L
            == lax.broadcasted_iota(jnp.int32, (BLK, RL), 0)).astype(jnp.float32)


def segTS_const():
    return (lax.broadcasted_iota(jnp.int32, (BLK, RS), 1) // S
            == lax.broadcasted_iota(jnp.int32, (BLK, RS), 0)).astype(jnp.float32)


def _tc_compute(qa, ur2, un, pi, weights, interpret=False):
    n_blocks = B // BLK
    row_spec = pl.BlockSpec((RL, D), lambda b: (b, 0))
    ur_spec = pl.BlockSpec((1, 1, RL), lambda b: (b, 0, 0))
    un_spec = pl.BlockSpec((RS, D), lambda b: (b, 0))
    pi_spec = pl.BlockSpec((BLK, D), lambda b: (b, 0))

    def w_spec(w):
        return pl.BlockSpec(w.shape, lambda b: tuple(0 for _ in w.shape))

    return pl.pallas_call(
        _tc_body,
        grid=(n_blocks,),
        in_specs=[row_spec, ur_spec, un_spec, pi_spec] +
                 [w_spec(w) for w in weights],
        out_specs=pl.BlockSpec((BLK, D), lambda b: (b, 0)),
        out_shape=jax.ShapeDtypeStruct((B, D), jnp.float32),
        compiler_params=pltpu.CompilerParams(
            dimension_semantics=("arbitrary",)),
        interpret=interpret,
    )(qa, ur2, un, pi, *weights)


def _prep_weights(embed_r_w, gv_w1, gv_b1, gv_w2, gv_b2,
                  attI_w1, attI_b1, attI_w2, attI_b2, attI_w3, attI_b3,
                  attS_w1, attS_b1, attS_w2, attS_b2, attS_w3, attS_b3,
                  mlp_w1, mlp_b1, mlp_w2, mlp_b2):
    emb_r8 = jnp.zeros((8, D), jnp.float32).at[:NR].set(embed_r_w)
    row = lambda v: v.reshape(1, -1)
    col = lambda v: v.reshape(-1, 1)
    return [
        emb_r8, gv_w1[:D], gv_w1[D:], col(gv_b1), gv_w2, col(gv_b2),
        attI_w1[:D], attI_w1[D:], col(attI_b1), attI_w2, col(attI_b2),
        attI_w3, row(attI_b3),
        attS_w1[:D], attS_w1[D:], col(attS_b1), attS_w2, col(attS_b2),
        attS_w3, row(attS_b3),
        mlp_w1[:D], mlp_w1[D:], row(mlp_b1), mlp_w2, row(mlp_b2),
    ]


def kernel(nodes_u, history_u_lists_batch, social_adj_lists_batch,
           history_ur_lists_batch,
           embed_u_w, embed_i_w, embed_r_w,
           gv_w1, gv_b1, gv_w2, gv_b2,
           attI_w1, attI_b1, attI_w2, attI_b2, attI_w3, attI_b3,
           attS_w1, attS_b1, attS_w2, attS_b2, attS_w3, attS_b3,
           mlp_w1, mlp_b1, mlp_w2, mlp_b2):
    idx_i = history_u_lists_batch.reshape(N_ITEM)
    idx_u = jnp.concatenate([
        social_adj_lists_batch.reshape(B * S), nodes_u,
        jnp.zeros((N_USER_PAD - B * S - B,), jnp.int32)])
    qa, gathered_u = _sc_gather(embed_i_w, embed_u_w, idx_i, idx_u)
    un = gathered_u[:B * S]
    pi = gathered_u[B * S:B * S + B]

    ur2 = history_ur_lists_batch.reshape(B // BLK, 1, RL)
    weights = _prep_weights(
        embed_r_w, gv_w1, gv_b1, gv_w2, gv_b2,
        attI_w1, attI_b1, attI_w2, attI_b2, attI_w3, attI_b3,
        attS_w1, attS_b1, attS_w2, attS_b2, attS_w3, attS_b3,
        mlp_w1, mlp_b1, mlp_w2, mlp_b2)
    return _tc_compute(qa, ur2, un, pi, weights)


# speedup vs baseline: 4.5378x; 1.0390x over previous
import functools

import jax
import jax.numpy as jnp
from jax import lax
from jax.experimental import pallas as pl
from jax.experimental.pallas import tpu as pltpu
from jax.experimental.pallas import tpu_sc as plsc

B, L, S, D = 1024, 200, 50, 128
NR = 6

NC, NS = 2, 16
NW = NC * NS
CH = 128

N_ITEM = B * L
N_USER_PAD = 53248
ITEM_PER_W = N_ITEM // NW
USER_PER_W = N_USER_PAD // NW


def _sc_gather_body(tab_i, tab_u, idx_i_hbm, idx_u_hbm, out_i, out_u,
                    idx_v, rows_v, sem):
    wid = lax.axis_index("s") * NC + lax.axis_index("c")

    def gather_loop(tab, idx_hbm, out, base, n_chunks):
        def body(j, carry):
            off = base + j * CH
            pltpu.sync_copy(idx_hbm.at[pl.ds(off, CH)], idx_v)
            pltpu.async_copy(tab.at[idx_v], rows_v, sem).wait()
            pltpu.sync_copy(rows_v, out.at[pl.ds(off, CH)])
            return carry
        lax.fori_loop(0, n_chunks, body, 0)

    gather_loop(tab_i, idx_i_hbm, out_i, wid * ITEM_PER_W, ITEM_PER_W // CH)
    gather_loop(tab_u, idx_u_hbm, out_u, wid * USER_PER_W, USER_PER_W // CH)


@jax.jit
def _sc_gather(embed_i_w, embed_u_w, idx_i, idx_u):
    mesh = plsc.VectorSubcoreMesh(core_axis_name="c", subcore_axis_name="s")
    return pl.kernel(
        _sc_gather_body,
        out_type=[
            jax.ShapeDtypeStruct((N_ITEM, D), jnp.float32),
            jax.ShapeDtypeStruct((N_USER_PAD, D), jnp.float32),
        ],
        mesh=mesh,
        scratch_types=[
            pltpu.VMEM((CH,), jnp.int32),
            pltpu.VMEM((CH, D), jnp.float32),
            pltpu.SemaphoreType.DMA,
        ],
    )(embed_i_w, embed_u_w, idx_i, idx_u)


BLK = 64
RL = BLK * L
RS = BLK * S


def _tc_body(qa_ref, ur_ref, un_ref, pi_ref,
             emb_r8_ref, gvw1a_ref, gvw1b_ref, gvb1_ref, gvw2_ref, gvb2_ref,
             aIw1a_ref, aIw1b_ref, aIb1_ref, aIw2_ref, aIb2_ref, aIw3_ref, aIb3_ref,
             aSw1a_ref, aSw1b_ref, aSb1_ref, aSw2_ref, aSb2_ref, aSw3_ref, aSb3_ref,
             mw1a_ref, mw1b_ref, mb1_ref, mw2_ref, mb2_ref,
             out_ref):
    f32 = jnp.float32
    bf16 = jnp.bfloat16

    def dt(lhs, rhs, lc, rc):
        return lax.dot_general(lhs.astype(bf16), rhs.astype(bf16),
                               (((lc,), (rc,)), ((), ())),
                               preferred_element_type=f32)

    qa = qa_ref[...]
    piB = pi_ref[...]
    urT = ur_ref[...].reshape(1, RL)

    onehotT = (urT == lax.broadcasted_iota(jnp.int32, (8, RL), 0)) \
        .astype(bf16)
    tr8 = jnp.dot(emb_r8_ref[...], gvw1b_ref[...],
                  preferred_element_type=f32)
    hT = jnp.maximum(dt(gvw1a_ref[...], qa, 0, 1)
                     + dt(tr8, onehotT, 0, 0)
                     + gvb1_ref[...], 0.0)
    xiaT = jnp.maximum(dt(gvw2_ref[...], hT, 0, 0)
                       + gvb2_ref[...], 0.0)

    def attention(featT_for_mlp, lc_feat, value_dot, segT,
                  w1a_ref, w1b_ref, b1_ref, w2_ref, b2_ref, w3_ref, b3_ref):
        piWT = dt(w1b_ref[...], piB, 0, 1)
        aT = jnp.maximum(dt(w1a_ref[...], featT_for_mlp, 0, lc_feat)
                         + dt(piWT, segT, 1, 0)
                         + b1_ref[...], 0.0)
        aT = jnp.maximum(dt(w2_ref[...], aT, 0, 0) + b2_ref[...], 0.0)
        logitT = dt(w3_ref[...], aT, 0, 0) + b3_ref[...]
        e = jnp.exp(logitT - jnp.max(logitT))
        AT = segT * e
        numer = value_dot(AT)
        den = lax.dot_general(AT, jnp.ones((AT.shape[1], 1), f32),
                              (((1,), (0,)), ((), ())),
                              preferred_element_type=f32)
        return numer / den

    segTI = segTI_const()
    segTS = segTS_const()
    hi_I = attention(xiaT, 0, lambda AT: dt(AT, xiaT, 1, 1), segTI,
                     aIw1a_ref, aIw1b_ref, aIb1_ref, aIw2_ref, aIb2_ref,
                     aIw3_ref, aIb3_ref)
    un = un_ref[...]
    hi_S = attention(un, 1, lambda AT: dt(AT, un, 1, 0), segTS,
                     aSw1a_ref, aSw1b_ref, aSb1_ref, aSw2_ref, aSb2_ref,
                     aSw3_ref, aSb3_ref)

    h2 = jnp.maximum(dt(hi_I, mw1a_ref[...], 1, 0)
                     + dt(hi_S, mw1b_ref[...], 1, 0)
                     + mb1_ref[...], 0.0)
    out_ref[...] = jnp.maximum(
        dt(h2, mw2_ref[...], 1, 0) + mb2_ref[...], 0.0)[:BLK]


def segTI_const():
    return (lax.broadcasted_iota(jnp.int32, (BLK, RL), 1) // L
            == lax.broadcasted_iota(jnp.int32, (BLK, RL), 0)).astype(jnp.float32)


def segTS_const():
    return (lax.broadcasted_iota(jnp.int32, (BLK, RS), 1) // S
            == lax.broadcasted_iota(jnp.int32, (BLK, RS), 0)).astype(jnp.float32)


def _tc_compute(qa, ur2, un, pi, weights, interpret=False):
    n_blocks = B // BLK
    row_spec = pl.BlockSpec((RL, D), lambda b: (b, 0))
    ur_spec = pl.BlockSpec((1, 1, RL), lambda b: (b, 0, 0))
    un_spec = pl.BlockSpec((RS, D), lambda b: (b, 0))
    pi_spec = pl.BlockSpec((BLK, D), lambda b: (b, 0))

    def w_spec(w):
        return pl.BlockSpec(w.shape, lambda b: tuple(0 for _ in w.shape))

    return pl.pallas_call(
        _tc_body,
        grid=(n_blocks,),
        in_specs=[row_spec, ur_spec, un_spec, pi_spec] +
                 [w_spec(w) for w in weights],
        out_specs=pl.BlockSpec((BLK, D), lambda b: (b, 0)),
        out_shape=jax.ShapeDtypeStruct((B, D), jnp.float32),
        compiler_params=pltpu.CompilerParams(
            dimension_semantics=("arbitrary",)),
        interpret=interpret,
    )(qa, ur2, un, pi, *weights)


def _prep_weights(embed_r_w, gv_w1, gv_b1, gv_w2, gv_b2,
                  attI_w1, attI_b1, attI_w2, attI_b2, attI_w3, attI_b3,
                  attS_w1, attS_b1, attS_w2, attS_b2, attS_w3, attS_b3,
                  mlp_w1, mlp_b1, mlp_w2, mlp_b2):
    emb_r8 = jnp.zeros((8, D), jnp.float32).at[:NR].set(embed_r_w)
    row = lambda v: v.reshape(1, -1)
    col = lambda v: v.reshape(-1, 1)
    return [
        emb_r8, gv_w1[:D], gv_w1[D:], col(gv_b1), gv_w2, col(gv_b2),
        attI_w1[:D], attI_w1[D:], col(attI_b1), attI_w2, col(attI_b2),
        attI_w3, row(attI_b3),
        attS_w1[:D], attS_w1[D:], col(attS_b1), attS_w2, col(attS_b2),
        attS_w3, row(attS_b3),
        mlp_w1[:D], mlp_w1[D:], row(mlp_b1), mlp_w2, row(mlp_b2),
    ]


def kernel(nodes_u, history_u_lists_batch, social_adj_lists_batch,
           history_ur_lists_batch,
           embed_u_w, embed_i_w, embed_r_w,
           gv_w1, gv_b1, gv_w2, gv_b2,
           attI_w1, attI_b1, attI_w2, attI_b2, attI_w3, attI_b3,
           attS_w1, attS_b1, attS_w2, attS_b2, attS_w3, attS_b3,
           mlp_w1, mlp_b1, mlp_w2, mlp_b2):
    idx_i = history_u_lists_batch.reshape(N_ITEM)
    idx_u = jnp.concatenate([
        social_adj_lists_batch.reshape(B * S), nodes_u,
        jnp.zeros((N_USER_PAD - B * S - B,), jnp.int32)])
    qa, gathered_u = _sc_gather(embed_i_w, embed_u_w, idx_i, idx_u)
    un = gathered_u[:B * S]
    pi = gathered_u[B * S:B * S + B]

    ur2 = history_ur_lists_batch.reshape(B // BLK, 1, RL)
    weights = _prep_weights(
        embed_r_w, gv_w1, gv_b1, gv_w2, gv_b2,
        attI_w1, attI_b1, attI_w2, attI_b2, attI_w3, attI_b3,
        attS_w1, attS_b1, attS_w2, attS_b2, attS_w3, attS_b3,
        mlp_w1, mlp_b1, mlp_w2, mlp_b2)
    return _tc_compute(qa, ur2, un, pi, weights)
